# baseline (speedup 1.0000x reference)
import functools

import jax
import jax.numpy as jnp
from jax import lax
from jax.experimental import pallas as pl
from jax.experimental.pallas import tpu as pltpu
from jax.experimental.pallas import tpu_sc as plsc

N = 10000
E = 320000
IN_DIM = 16
OUT_DIM = 16
EDGE_DIM = 16
HIDDEN = 64

_NC = 2
_NS = 16
_NW = _NC * _NS
_EPW = E // _NW
_CHUNK = 2000
_NCHUNK = _EPW // _CHUNK

def _sc_mesh():
    return plsc.VectorSubcoreMesh(
        core_axis_name="c", subcore_axis_name="s",
        num_cores=_NC, num_subcores=_NS,
    )


@functools.cache
def _gather_kernel():
    @functools.partial(
        pl.kernel,
        out_type=(
            jax.ShapeDtypeStruct((E, IN_DIM), jnp.float32),
            jax.ShapeDtypeStruct((E, EDGE_DIM), jnp.float32),
        ),
        mesh=_sc_mesh(),
        scratch_types=[
            pltpu.VMEM((_CHUNK,), jnp.int32),
            pltpu.VMEM((_CHUNK, IN_DIM), jnp.float32),
            pltpu.VMEM((_CHUNK, EDGE_DIM), jnp.float32),
            pltpu.SemaphoreType.DMA,
        ],
        compiler_params=pltpu.CompilerParams(use_tc_tiling_on_sc=False),
    )
    def _gather_k(nf_hbm, ei_hbm, ef_hbm, out_hbm, ef_out, idx_v, rows_v,
                  ef_v, sem):
        wid = lax.axis_index("s") * _NC + lax.axis_index("c")
        base_w = wid * _EPW
        for c in range(_NCHUNK):
            base = base_w + c * _CHUNK
            pltpu.sync_copy(ei_hbm.at[0, pl.ds(base, _CHUNK)], idx_v)
            pltpu.async_copy(nf_hbm.at[idx_v], rows_v, sem).wait()
            pltpu.sync_copy(rows_v, out_hbm.at[pl.ds(base, _CHUNK)])
            pltpu.sync_copy(ef_hbm.at[pl.ds(base, _CHUNK)], ef_v)
            pltpu.sync_copy(ef_v, ef_out.at[pl.ds(base, _CHUNK)])

    return _gather_k


@functools.cache
def _scatter_kernel():
    @functools.partial(
        pl.kernel,
        out_type=(
            jax.ShapeDtypeStruct((_NC, N, OUT_DIM), jnp.float32),
            jax.ShapeDtypeStruct((_NC, N, OUT_DIM), jnp.float32),
        ),
        mesh=_sc_mesh(),
        scratch_types=[
            pltpu.VMEM_SHARED((N, OUT_DIM), jnp.float32),
            pltpu.VMEM_SHARED((N, OUT_DIM), jnp.float32),
            pltpu.VMEM((_CHUNK,), jnp.int32),
            pltpu.VMEM((_CHUNK, OUT_DIM), jnp.float32),
            pltpu.VMEM((_CHUNK, OUT_DIM), jnp.float32),
        ],
        compiler_params=pltpu.CompilerParams(use_tc_tiling_on_sc=False),
    )
    def _scatter_k(msg_hbm, ei_hbm, zeros_hbm, ones_hbm,
                   agg_out, cnt_out, agg_sh, cnt_sh, idx_v, msg_v, ones_v):
        cid = lax.axis_index("c")
        sid = lax.axis_index("s")
        wid = sid * _NC + cid
        pltpu.sync_copy(ones_hbm, ones_v)

        @pl.when(sid == 0)
        def _init():
            pltpu.sync_copy(zeros_hbm, agg_sh)
            pltpu.sync_copy(zeros_hbm, cnt_sh)

        plsc.subcore_barrier()
        base_w = wid * _EPW
        for c in range(_NCHUNK):
            base = base_w + c * _CHUNK
            pltpu.sync_copy(ei_hbm.at[1, pl.ds(base, _CHUNK)], idx_v)
            pltpu.sync_copy(msg_hbm.at[pl.ds(base, _CHUNK)], msg_v)
            pltpu.sync_copy(msg_v, agg_sh.at[idx_v], add=True)
            pltpu.sync_copy(ones_v, cnt_sh.at[idx_v], add=True)
        plsc.subcore_barrier()

        @pl.when(sid == 0)
        def _flush():
            pltpu.sync_copy(agg_sh, agg_out.at[cid])
            pltpu.sync_copy(cnt_sh, cnt_out.at[cid])

    return _scatter_k


_BE = 12800


def _mlp_body(ef_ref, xs_ref, w1_ref, b1_ref, w2_ref, b2_ref, r_ref, s_ref,
              out_ref):
    ef = ef_ref[...].astype(jnp.bfloat16)
    xs = xs_ref[...].astype(jnp.bfloat16)
    h = jnp.dot(ef, w1_ref[...], preferred_element_type=jnp.float32)
    h = h + b1_ref[...]
    h = jnp.where(h >= 0, h, 0.01 * h).astype(jnp.bfloat16)
    w = jnp.dot(h, w2_ref[...], preferred_element_type=jnp.float32)
    w = w + b2_ref[...]
    xrep = jnp.dot(xs, r_ref[...], preferred_element_type=jnp.float32)
    p = (w * xrep).astype(jnp.bfloat16)
    out_ref[...] = jnp.dot(p, s_ref[...], preferred_element_type=jnp.float32)


def _edge_mlp(edge_feat_p, x_src_p, W1, b1, W2, b2):
    grid = (E // _BE,)
    eye8 = jnp.eye(8, dtype=jnp.float32)
    rmat = jnp.repeat(jnp.eye(IN_DIM, dtype=jnp.float32), OUT_DIM, axis=1)
    smat = jnp.tile(jnp.eye(OUT_DIM, dtype=jnp.float32), (IN_DIM, 1))
    w1bd = jnp.kron(eye8, W1).astype(jnp.bfloat16)
    w2bd = jnp.kron(eye8, W2).astype(jnp.bfloat16)
    rbd = jnp.kron(eye8, rmat).astype(jnp.bfloat16)
    sbd = jnp.kron(eye8, smat).astype(jnp.bfloat16)
    b1p = jnp.tile(b1, 8).reshape(1, 8 * HIDDEN)
    b2p = jnp.tile(b2, 8).reshape(1, 8 * IN_DIM * OUT_DIM)
    b8 = _BE // 8
    return pl.pallas_call(
        _mlp_body,
        grid=grid,
        in_specs=[
            pl.BlockSpec((b8, 8 * EDGE_DIM), lambda i: (i, 0)),
            pl.BlockSpec((b8, 8 * IN_DIM), lambda i: (i, 0)),
            pl.BlockSpec((8 * EDGE_DIM, 8 * HIDDEN), lambda i: (0, 0)),
            pl.BlockSpec((1, 8 * HIDDEN), lambda i: (0, 0)),
            pl.BlockSpec((8 * HIDDEN, 8 * IN_DIM * OUT_DIM), lambda i: (0, 0)),
            pl.BlockSpec((1, 8 * IN_DIM * OUT_DIM), lambda i: (0, 0)),
            pl.BlockSpec((8 * IN_DIM, 8 * IN_DIM * OUT_DIM), lambda i: (0, 0)),
            pl.BlockSpec((8 * IN_DIM * OUT_DIM, 8 * OUT_DIM), lambda i: (0, 0)),
        ],
        out_specs=pl.BlockSpec((b8, 8 * OUT_DIM), lambda i: (i, 0)),
        out_shape=jax.ShapeDtypeStruct((E // 8, 8 * OUT_DIM), jnp.float32),
    )(edge_feat_p, x_src_p, w1bd, b1p, w2bd, b2p, rbd, sbd)


def _final_body(agg_ref, cnt_ref, nf_ref, root_ref, bias_ref, out_ref):
    agg = agg_ref[0] + agg_ref[1]
    cnt = cnt_ref[0] + cnt_ref[1]
    mean = agg / jnp.maximum(cnt, 1.0)
    out = mean + jnp.dot(nf_ref[...], root_ref[...],
                         preferred_element_type=jnp.float32) + bias_ref[...]
    out_ref[...] = jnp.where(out >= 0, out, 0.01 * out)


def _finalize(agg2p, cnt2p, node_feat, root, bias):
    rootbd = jnp.kron(jnp.eye(8, dtype=jnp.float32), root)
    biasp = jnp.tile(bias, 8).reshape(1, 8 * OUT_DIM)
    outp = pl.pallas_call(
        _final_body,
        out_shape=jax.ShapeDtypeStruct((N // 8, 8 * OUT_DIM), jnp.float32),
    )(agg2p, cnt2p, node_feat.reshape(N // 8, 8 * IN_DIM), rootbd, biasp)
    return outp.reshape(N, OUT_DIM)


def kernel(node_feat, edge_feat, W1, b1, W2, b2, root, bias, edge_index,
           batch_index, num_sampled_nodes_per_hop, num_sampled_edges_per_hop):
    x_src, ef_lin = _gather_kernel()(node_feat, edge_index, edge_feat)
    msg_p = _edge_mlp(ef_lin.reshape(E // 8, 8 * EDGE_DIM),
                      x_src.reshape(E // 8, 8 * IN_DIM),
                      W1, b1, W2, b2)
    zeros = jnp.zeros((N, OUT_DIM), jnp.float32)
    ones = jnp.ones((_CHUNK, OUT_DIM), jnp.float32)
    agg2, cnt2 = _scatter_kernel()(msg_p.reshape(E, OUT_DIM), edge_index,
                                   zeros, ones)
    out = _finalize(agg2.reshape(_NC, N // 8, 8 * OUT_DIM),
                    cnt2.reshape(_NC, N // 8, 8 * OUT_DIM),
                    node_feat, root, bias)
    return (out, edge_index, edge_feat)


# speedup vs baseline: 1.0688x; 1.0688x over previous
import functools

import jax
import jax.numpy as jnp
from jax import lax
from jax.experimental import pallas as pl
from jax.experimental.pallas import tpu as pltpu
from jax.experimental.pallas import tpu_sc as plsc

N = 10000
E = 320000
IN_DIM = 16
OUT_DIM = 16
EDGE_DIM = 16
HIDDEN = 64

_NC = 2
_NS = 16
_NW = _NC * _NS
_EPW = E // _NW
_CHUNK = 2000
_NCHUNK = _EPW // _CHUNK

def _sc_mesh():
    return plsc.VectorSubcoreMesh(
        core_axis_name="c", subcore_axis_name="s",
        num_cores=_NC, num_subcores=_NS,
    )


@functools.cache
def _gather_kernel():
    @functools.partial(
        pl.kernel,
        out_type=jax.ShapeDtypeStruct((E, IN_DIM), jnp.float32),
        mesh=_sc_mesh(),
        scratch_types=[
            pltpu.VMEM((_CHUNK,), jnp.int32),
            pltpu.VMEM((_CHUNK, IN_DIM), jnp.float32),
            pltpu.SemaphoreType.DMA,
        ],
        compiler_params=pltpu.CompilerParams(use_tc_tiling_on_sc=False),
    )
    def _gather_k(nf_hbm, ei_hbm, out_hbm, idx_v, rows_v, sem):
        wid = lax.axis_index("s") * _NC + lax.axis_index("c")
        base_w = wid * _EPW
        for c in range(_NCHUNK):
            base = base_w + c * _CHUNK
            pltpu.sync_copy(ei_hbm.at[0, pl.ds(base, _CHUNK)], idx_v)
            pltpu.async_copy(nf_hbm.at[idx_v], rows_v, sem).wait()
            pltpu.sync_copy(rows_v, out_hbm.at[pl.ds(base, _CHUNK)])

    return _gather_k


@functools.cache
def _scatter_kernel():
    @functools.partial(
        pl.kernel,
        out_type=(
            jax.ShapeDtypeStruct((_NC, N, OUT_DIM), jnp.float32),
            jax.ShapeDtypeStruct((_NC, N, OUT_DIM), jnp.float32),
        ),
        mesh=_sc_mesh(),
        scratch_types=[
            pltpu.VMEM_SHARED((N, OUT_DIM), jnp.float32),
            pltpu.VMEM_SHARED((N, OUT_DIM), jnp.float32),
            pltpu.VMEM((_CHUNK,), jnp.int32),
            pltpu.VMEM((_CHUNK, OUT_DIM), jnp.float32),
            pltpu.VMEM((_CHUNK, OUT_DIM), jnp.float32),
        ],
        compiler_params=pltpu.CompilerParams(use_tc_tiling_on_sc=False),
    )
    def _scatter_k(msg_hbm, ei_hbm, zeros_hbm, ones_hbm,
                   agg_out, cnt_out, agg_sh, cnt_sh, idx_v, msg_v, ones_v):
        cid = lax.axis_index("c")
        sid = lax.axis_index("s")
        wid = sid * _NC + cid
        pltpu.sync_copy(ones_hbm, ones_v)

        @pl.when(sid == 0)
        def _init():
            pltpu.sync_copy(zeros_hbm, agg_sh)
            pltpu.sync_copy(zeros_hbm, cnt_sh)

        plsc.subcore_barrier()
        base_w = wid * _EPW
        for c in range(_NCHUNK):
            base = base_w + c * _CHUNK
            pltpu.sync_copy(ei_hbm.at[1, pl.ds(base, _CHUNK)], idx_v)
            pltpu.sync_copy(msg_hbm.at[pl.ds(base, _CHUNK)], msg_v)
            pltpu.sync_copy(msg_v, agg_sh.at[idx_v], add=True)
            pltpu.sync_copy(ones_v, cnt_sh.at[idx_v], add=True)
        plsc.subcore_barrier()

        @pl.when(sid == 0)
        def _flush():
            pltpu.sync_copy(agg_sh, agg_out.at[cid])
            pltpu.sync_copy(cnt_sh, cnt_out.at[cid])

    return _scatter_k


_BE = 12800


def _mlp_body(ef_ref, xs_ref, w1_ref, b1_ref, w2_ref, b2_ref, r_ref, s_ref,
              out_ref):
    ef = ef_ref[...].astype(jnp.bfloat16)
    xs = xs_ref[...].astype(jnp.bfloat16)
    h = jnp.dot(ef, w1_ref[...], preferred_element_type=jnp.float32)
    h = h + b1_ref[...]
    h = jnp.where(h >= 0, h, 0.01 * h).astype(jnp.bfloat16)
    w = jnp.dot(h, w2_ref[...], preferred_element_type=jnp.float32)
    w = w + b2_ref[...]
    xrep = jnp.dot(xs, r_ref[...], preferred_element_type=jnp.float32)
    p = (w * xrep).astype(jnp.bfloat16)
    out_ref[...] = jnp.dot(p, s_ref[...], preferred_element_type=jnp.float32)


def _edge_mlp(edge_feat_p, x_src_p, W1, b1, W2, b2):
    grid = (E // _BE,)
    eye8 = jnp.eye(8, dtype=jnp.float32)
    rmat = jnp.repeat(jnp.eye(IN_DIM, dtype=jnp.float32), OUT_DIM, axis=1)
    smat = jnp.tile(jnp.eye(OUT_DIM, dtype=jnp.float32), (IN_DIM, 1))
    w1bd = jnp.kron(eye8, W1).astype(jnp.bfloat16)
    w2bd = jnp.kron(eye8, W2).astype(jnp.bfloat16)
    rbd = jnp.kron(eye8, rmat).astype(jnp.bfloat16)
    sbd = jnp.kron(eye8, smat).astype(jnp.bfloat16)
    b1p = jnp.tile(b1, 8).reshape(1, 8 * HIDDEN)
    b2p = jnp.tile(b2, 8).reshape(1, 8 * IN_DIM * OUT_DIM)
    b8 = _BE // 8
    return pl.pallas_call(
        _mlp_body,
        grid=grid,
        in_specs=[
            pl.BlockSpec((b8, 8 * EDGE_DIM), lambda i: (i, 0)),
            pl.BlockSpec((b8, 8 * IN_DIM), lambda i: (i, 0)),
            pl.BlockSpec((8 * EDGE_DIM, 8 * HIDDEN), lambda i: (0, 0)),
            pl.BlockSpec((1, 8 * HIDDEN), lambda i: (0, 0)),
            pl.BlockSpec((8 * HIDDEN, 8 * IN_DIM * OUT_DIM), lambda i: (0, 0)),
            pl.BlockSpec((1, 8 * IN_DIM * OUT_DIM), lambda i: (0, 0)),
            pl.BlockSpec((8 * IN_DIM, 8 * IN_DIM * OUT_DIM), lambda i: (0, 0)),
            pl.BlockSpec((8 * IN_DIM * OUT_DIM, 8 * OUT_DIM), lambda i: (0, 0)),
        ],
        out_specs=pl.BlockSpec((b8, 8 * OUT_DIM), lambda i: (i, 0)),
        out_shape=jax.ShapeDtypeStruct((E // 8, 8 * OUT_DIM), jnp.float32),
    )(edge_feat_p, x_src_p, w1bd, b1p, w2bd, b2p, rbd, sbd)


def _final_body(agg_ref, cnt_ref, nf_ref, root_ref, bias_ref, out_ref):
    agg = agg_ref[0] + agg_ref[1]
    cnt = cnt_ref[0] + cnt_ref[1]
    mean = agg / jnp.maximum(cnt, 1.0)
    out = mean + jnp.dot(nf_ref[...], root_ref[...],
                         preferred_element_type=jnp.float32) + bias_ref[...]
    out_ref[...] = jnp.where(out >= 0, out, 0.01 * out)


def _finalize(agg2p, cnt2p, node_feat, root, bias):
    rootbd = jnp.kron(jnp.eye(8, dtype=jnp.float32), root)
    biasp = jnp.tile(bias, 8).reshape(1, 8 * OUT_DIM)
    outp = pl.pallas_call(
        _final_body,
        out_shape=jax.ShapeDtypeStruct((N // 8, 8 * OUT_DIM), jnp.float32),
    )(agg2p, cnt2p, node_feat.reshape(N // 8, 8 * IN_DIM), rootbd, biasp)
    return outp.reshape(N, OUT_DIM)


def kernel(node_feat, edge_feat, W1, b1, W2, b2, root, bias, edge_index,
           batch_index, num_sampled_nodes_per_hop, num_sampled_edges_per_hop):
    x_src = _gather_kernel()(node_feat, edge_index)
    msg_p = _edge_mlp(edge_feat.reshape(E // 8, 8 * EDGE_DIM),
                      x_src.reshape(E // 8, 8 * IN_DIM),
                      W1, b1, W2, b2)
    zeros = jnp.zeros((N, OUT_DIM), jnp.float32)
    ones = jnp.ones((_CHUNK, OUT_DIM), jnp.float32)
    agg2, cnt2 = _scatter_kernel()(msg_p.reshape(E, OUT_DIM), edge_index,
                                   zeros, ones)
    out = _finalize(agg2.reshape(_NC, N // 8, 8 * OUT_DIM),
                    cnt2.reshape(_NC, N // 8, 8 * OUT_DIM),
                    node_feat, root, bias)
    return (out, edge_index, edge_feat)
